# baseline (speedup 1.0000x reference)
import functools

import jax
import jax.numpy as jnp
from jax import lax
from jax.experimental import pallas as pl
from jax.experimental.pallas import tpu as pltpu
from jax.experimental.pallas import tpu_sc as plsc

N = 10000
E = 160000
OUT = 3
NZF = float(N * OUT)

NC, NS = 2, 16
NW = NC * NS
CH = 128
EPW = 5120
EPAD = NW * EPW
NCHUNK = EPW // CH
ACC = 10240
RPT = ACC // NS

NB = 1000
EB = 2048


def _ln(h, s, t):
    m = jnp.mean(h, axis=-1, keepdims=True)
    v = jnp.mean((h - m) ** 2, axis=-1, keepdims=True)
    return (h - m) / jnp.sqrt(v + 1e-5) * s + t


def _full(shape):
    return pl.BlockSpec(shape, lambda i: (0,) * len(shape))


def _encode_body(xin, w0, b0, w1, b1, w2, b2, s, t, out):
    x = xin[...]
    h = jnp.maximum(x @ w0[...] + b0[...], 0.0)
    h = jnp.maximum(h @ w1[...] + b1[...], 0.0)
    h = h @ w2[...] + b2[...]
    xln = _ln(h, s[...], t[...])
    pos = x[:, 124:127]
    out[...] = jnp.concatenate(
        [xln, pos, jnp.zeros((xln.shape[0], 61), jnp.float32)], axis=-1)


def _make_encode(interpret=False):
    return pl.pallas_call(
        _encode_body,
        grid=(N // NB,),
        in_specs=[pl.BlockSpec((NB, 128), lambda i: (i, 0))]
        + [_full(s) for s in [(128, 64), (1, 64), (64, 64), (1, 64),
                              (64, 64), (1, 64), (1, 64), (1, 64)]],
        out_specs=pl.BlockSpec((NB, 128), lambda i: (i, 0)),
        out_shape=jax.ShapeDtypeStruct((N, 128), jnp.float32),
        interpret=interpret,
    )


def _edge0_body(gs_, gd_,
                we0p, wdist, be0, we1, be1, we2, be2, es, et,
                mwe, mwxs, mwxd, mb1, mw2, mb2, mw3, mb3, ms, mt,
                out):
    gs = gs_[...]
    gd = gd_[...]
    xs, xd = gs[:, :64], gd[:, :64]
    d = gd[:, 64:80] - gs[:, 64:80]
    dist = jnp.sqrt(jnp.sum(d * d, axis=-1, keepdims=True) + 1e-12)
    h = d @ we0p[...] + dist * wdist[...] + be0[...]
    h = jnp.maximum(h, 0.0)
    h = jnp.maximum(h @ we1[...] + be1[...], 0.0)
    e0 = _ln(h @ we2[...] + be2[...], es[...], et[...])
    h = e0 @ mwe[...] + xs @ mwxs[...] + xd @ mwxd[...] + mb1[...]
    h = jnp.maximum(h, 0.0)
    h = jnp.maximum(h @ mw2[...] + mb2[...], 0.0)
    ef = _ln(h @ mw3[...] + mb3[...], ms[...], mt[...])
    e1 = e0 + ef
    row = pl.program_id(0) * EB + lax.broadcasted_iota(jnp.int32, (EB, 1), 0)
    e1 = jnp.concatenate([e1, jnp.zeros((EB, 64), jnp.float32)], axis=-1)
    out[...] = jnp.where(row < E, e1, 0.0)


def _make_edge0(interpret=False):
    wshapes = [(16, 64), (1, 64), (1, 64), (64, 64), (1, 64), (64, 64),
               (1, 64), (1, 64), (1, 64),
               (64, 64), (64, 64), (64, 64), (1, 64), (64, 64), (1, 64),
               (64, 64), (1, 64), (1, 64), (1, 64)]
    return pl.pallas_call(
        _edge0_body,
        grid=(EPAD // EB,),
        in_specs=[pl.BlockSpec((EB, 128), lambda i: (i, 0)),
                  pl.BlockSpec((EB, 128), lambda i: (i, 0))]
        + [_full(s) for s in wshapes],
        out_specs=pl.BlockSpec((EB, 128), lambda i: (i, 0)),
        out_shape=jax.ShapeDtypeStruct((EPAD, 128), jnp.float32),
        interpret=interpret,
    )


def _edge1_body(e, gs_, gd_, mwe, mwxs, mwxd, mb1, mw2, mb2, mw3, mb3, ms, mt,
                out):
    ein = e[...][:, :64]
    xs, xd = gs_[...][:, :64], gd_[...][:, :64]
    h = ein @ mwe[...] + xs @ mwxs[...] + xd @ mwxd[...] + mb1[...]
    h = jnp.maximum(h, 0.0)
    h = jnp.maximum(h @ mw2[...] + mb2[...], 0.0)
    ef = _ln(h @ mw3[...] + mb3[...], ms[...], mt[...])
    e1 = ein + ef
    row = pl.program_id(0) * EB + lax.broadcasted_iota(jnp.int32, (EB, 1), 0)
    e1 = jnp.concatenate([e1, jnp.zeros((EB, 64), jnp.float32)], axis=-1)
    out[...] = jnp.where(row < E, e1, 0.0)


def _make_edge1(interpret=False):
    wshapes = [(64, 64), (64, 64), (64, 64), (1, 64), (64, 64), (1, 64),
               (64, 64), (1, 64), (1, 64), (1, 64)]
    return pl.pallas_call(
        _edge1_body,
        grid=(EPAD // EB,),
        in_specs=[pl.BlockSpec((EB, 128), lambda i: (i, 0)),
                  pl.BlockSpec((EB, 128), lambda i: (i, 0)),
                  pl.BlockSpec((EB, 128), lambda i: (i, 0))]
        + [_full(s) for s in wshapes],
        out_specs=pl.BlockSpec((EB, 128), lambda i: (i, 0)),
        out_shape=jax.ShapeDtypeStruct((EPAD, 128), jnp.float32),
        interpret=interpret,
    )


def _node0_body(x, a0, a1, wx, wa, b1, w2, b2, w3, b3, s, t, out):
    xin = x[...][:, :64]
    a = (a0[...][0] + a1[...][0])[:, :64]
    h = jnp.maximum(xin @ wx[...] + a @ wa[...] + b1[...], 0.0)
    h = jnp.maximum(h @ w2[...] + b2[...], 0.0)
    nf = _ln(h @ w3[...] + b3[...], s[...], t[...])
    x1 = xin + nf
    out[...] = jnp.concatenate(
        [x1, jnp.zeros((x1.shape[0], 64), jnp.float32)], axis=-1)


def _make_node0(interpret=False):
    wshapes = [(64, 64), (64, 64), (1, 64), (64, 64), (1, 64), (64, 64),
               (1, 64), (1, 64), (1, 64)]
    return pl.pallas_call(
        _node0_body,
        grid=(N // NB,),
        in_specs=[pl.BlockSpec((NB, 128), lambda i: (i, 0)),
                  pl.BlockSpec((1, NB, 128), lambda i: (0, i, 0)),
                  pl.BlockSpec((1, NB, 128), lambda i: (1, i, 0))]
        + [_full(s) for s in wshapes],
        out_specs=pl.BlockSpec((NB, 128), lambda i: (i, 0)),
        out_shape=jax.ShapeDtypeStruct((N, 128), jnp.float32),
        interpret=interpret,
    )


def _node1_body(x, a0, a1, tar,
                wx, wa, b1, w2, b2, w3, b3, s, t,
                dw0, db0, dw1, db1, dw2p, db2p,
                outp, loss):
    xin = x[...][:, :64]
    a = (a0[...][0] + a1[...][0])[:, :64]
    h = jnp.maximum(xin @ wx[...] + a @ wa[...] + b1[...], 0.0)
    h = jnp.maximum(h @ w2[...] + b2[...], 0.0)
    nf = _ln(h @ w3[...] + b3[...], s[...], t[...])
    x2 = xin + nf
    h = jnp.maximum(x2 @ dw0[...] + db0[...], 0.0)
    h = jnp.maximum(h @ dw1[...] + db1[...], 0.0)
    o = h @ dw2p[...] + db2p[...]
    outp[...] = o
    bs = jnp.sum((o - tar[...]) ** 2)
    i = pl.program_id(0)
    acc = jnp.where(i == 0, 0.0, loss[...]) + bs
    loss[...] = jnp.where(i == pl.num_programs(0) - 1, acc / NZF, acc)


def _make_node1(interpret=False):
    wshapes = [(64, 64), (64, 64), (1, 64), (64, 64), (1, 64), (64, 64),
               (1, 64), (1, 64), (1, 64),
               (64, 64), (1, 64), (64, 64), (1, 64), (64, 128), (1, 128)]
    return pl.pallas_call(
        _node1_body,
        grid=(N // NB,),
        in_specs=[pl.BlockSpec((NB, 128), lambda i: (i, 0)),
                  pl.BlockSpec((1, NB, 128), lambda i: (0, i, 0)),
                  pl.BlockSpec((1, NB, 128), lambda i: (1, i, 0)),
                  pl.BlockSpec((NB, 128), lambda i: (i, 0))]
        + [_full(s) for s in wshapes],
        out_specs=[pl.BlockSpec((NB, 128), lambda i: (i, 0)),
                   pl.BlockSpec((1, 1), lambda i: (0, 0))],
        out_shape=[jax.ShapeDtypeStruct((N, 128), jnp.float32),
                   jax.ShapeDtypeStruct((1, 1), jnp.float32)],
        interpret=interpret,
    )


@functools.cache
def _sc_mesh():
    return plsc.VectorSubcoreMesh(core_axis_name="c", subcore_axis_name="s",
                                  num_cores=NC, num_subcores=NS)


def _gather_x_body(src2, dst2, xt, oxs, oxd, idx_s, idx_d,
                   bxs0, bxd0, bxs1, bxd1, sr0, sw0, sr1, sw1):
    c = lax.axis_index("c")
    s = lax.axis_index("s")
    wid = s * NC + c
    pltpu.sync_copy(src2.at[pl.ds(wid * NCHUNK, NCHUNK)], idx_s)
    pltpu.sync_copy(dst2.at[pl.ds(wid * NCHUNK, NCHUNK)], idx_d)
    base = wid * EPW

    bufs = ((bxs0, bxd0, sr0, sw0), (bxs1, bxd1, sr1, sw1))

    for b in range(2):
        bx, bd, sr, _ = bufs[b]
        pltpu.async_copy(xt.at[idx_s.at[b]], bx, sr)
        pltpu.async_copy(xt.at[idx_d.at[b]], bd, sr)

    @pl.loop(0, NCHUNK, step=2)
    def _(j0):
        for b in range(2):
            j = j0 + b
            bx, bd, sr, sw = bufs[b]
            pltpu.make_async_copy(xt.at[idx_s.at[j]], bx, sr).wait()
            pltpu.make_async_copy(xt.at[idx_d.at[j]], bd, sr).wait()
            r0 = base + j * CH
            pltpu.async_copy(bx, oxs.at[pl.ds(r0, CH)], sw)
            pltpu.async_copy(bd, oxd.at[pl.ds(r0, CH)], sw)

            @pl.when(j + 2 < NCHUNK)
            def _():
                pltpu.make_async_copy(bx, oxs.at[pl.ds(r0, CH)], sw).wait()
                pltpu.make_async_copy(bd, oxd.at[pl.ds(r0, CH)], sw).wait()
                pltpu.async_copy(xt.at[idx_s.at[j + 2]], bx, sr)
                pltpu.async_copy(xt.at[idx_d.at[j + 2]], bd, sr)

    for b in range(2):
        bx, bd, _, sw = bufs[b]
        pltpu.make_async_copy(bx, oxs.at[pl.ds(0, CH)], sw).wait()
        pltpu.make_async_copy(bd, oxd.at[pl.ds(0, CH)], sw).wait()


@functools.cache
def _gather_x():
    return pl.kernel(
        _gather_x_body,
        out_type=[jax.ShapeDtypeStruct((EPAD, 128), jnp.float32),
                  jax.ShapeDtypeStruct((EPAD, 128), jnp.float32)],
        mesh=_sc_mesh(),
        scratch_types=[pltpu.VMEM((NCHUNK, CH), jnp.int32),
                       pltpu.VMEM((NCHUNK, CH), jnp.int32),
                       pltpu.VMEM((CH, 128), jnp.float32),
                       pltpu.VMEM((CH, 128), jnp.float32),
                       pltpu.VMEM((CH, 128), jnp.float32),
                       pltpu.VMEM((CH, 128), jnp.float32),
                       pltpu.SemaphoreType.DMA,
                       pltpu.SemaphoreType.DMA,
                       pltpu.SemaphoreType.DMA,
                       pltpu.SemaphoreType.DMA],
    )


def _scatter_body(e2, dst2, zeros_hbm, out, idxc, ebuf, acc):
    c = lax.axis_index("c")
    s = lax.axis_index("s")
    wid = c * NS + s
    rows0 = s * RPT
    pltpu.sync_copy(zeros_hbm.at[pl.ds(rows0, RPT)], acc.at[pl.ds(rows0, RPT)])
    plsc.subcore_barrier()
    base = wid * EPW

    def body(j, carry):
        pltpu.sync_copy(dst2.at[wid * NCHUNK + j], idxc)
        pltpu.sync_copy(e2.at[pl.ds(base + j * CH, CH)], ebuf)
        pltpu.sync_copy(ebuf, acc.at[idxc], add=True)
        return carry

    lax.fori_loop(0, NCHUNK, body, 0)
    plsc.subcore_barrier()
    pltpu.sync_copy(acc.at[pl.ds(rows0, RPT)], out.at[c, pl.ds(rows0, RPT)])


@functools.cache
def _scatter():
    return pl.kernel(
        _scatter_body,
        out_type=jax.ShapeDtypeStruct((NC, ACC, 128), jnp.float32),
        mesh=_sc_mesh(),
        scratch_types=[pltpu.VMEM((CH,), jnp.int32),
                       pltpu.VMEM((CH, 128), jnp.float32),
                       pltpu.VMEM_SHARED((ACC, 128), jnp.float32)],
    )


def kernel(m_idx, m_gs, node_in, node_tar, params):
    x0in = node_in[0]
    tar = node_tar[0]
    src = m_gs[0, 0].astype(jnp.int32)
    dst = m_gs[0, 1].astype(jnp.int32)
    src2 = jnp.pad(src, (0, EPAD - E)).reshape(EPAD // CH, CH)
    dst2 = jnp.pad(dst, (0, EPAD - E)).reshape(EPAD // CH, CH)

    tar_pad = jnp.pad(tar, ((0, 0), (0, 125)))
    zeros_acc = jnp.zeros((ACC, 128), jnp.float32)

    p = params

    def lyr(mp, i):
        return mp["layers"][i]

    def b2(x):
        return x.reshape(1, -1)

    enc = p["encode"]
    (ew0, eb0), (ew1, eb1), (ew2, eb2) = enc["layers"]
    es, et = enc["ln"]
    x0 = _make_encode()(x0in, ew0, b2(eb0), ew1, b2(eb1), ew2, b2(eb2),
                        b2(es), b2(et))

    xs0, xd0 = _gather_x()(src2, dst2, x0)

    ee = p["edge_enc"]
    (gw0, gb0), (gw1, gb1), (gw2, gb2) = ee["layers"]
    gs, gt = ee["ln"]
    we0p = jnp.pad(gw0[:3], ((0, 13), (0, 0)))
    wdist = gw0[3:4]
    m0 = lyr(p, 0)["edge"]
    (aw1, ab1), (aw2, ab2), (aw3, ab3) = m0["layers"]
    as_, at_ = m0["ln"]
    e1 = _make_edge0()(xs0, xd0,
                       we0p, wdist, b2(gb0), gw1, b2(gb1), gw2, b2(gb2),
                       b2(gs), b2(gt),
                       aw1[:64], aw1[64:128], aw1[128:], b2(ab1),
                       aw2, b2(ab2), aw3, b2(ab3), b2(as_), b2(at_))

    agg0 = _scatter()(e1, dst2, zeros_acc)

    n0 = lyr(p, 0)["node"]
    (nw1, nb1), (nw2, nb2), (nw3, nb3) = n0["layers"]
    ns_, nt_ = n0["ln"]
    x1 = _make_node0()(x0, agg0, agg0,
                       nw1[:64], nw1[64:], b2(nb1), nw2, b2(nb2),
                       nw3, b2(nb3), b2(ns_), b2(nt_))

    xs1, xd1 = _gather_x()(src2, dst2, x1)
    m1 = lyr(p, 1)["edge"]
    (cw1, cb1), (cw2, cb2), (cw3, cb3) = m1["layers"]
    cs_, ct_ = m1["ln"]
    e2 = _make_edge1()(e1, xs1, xd1,
                       cw1[:64], cw1[64:128], cw1[128:], b2(cb1),
                       cw2, b2(cb2), cw3, b2(cb3), b2(cs_), b2(ct_))
    agg1 = _scatter()(e2, dst2, zeros_acc)

    n1 = lyr(p, 1)["node"]
    (mw1, mb1), (mw2v, mb2v), (mw3v, mb3v) = n1["layers"]
    ms_, mt_ = n1["ln"]
    dec = p["decode"]
    (dw0, db0), (dw1, db1), (dw2, db2v) = dec["layers"]
    dw2p = jnp.pad(dw2, ((0, 0), (0, 125)))
    db2p = jnp.pad(db2v.reshape(1, -1), ((0, 0), (0, 125)))
    outp, loss = _make_node1()(x1, agg1, agg1, tar_pad,
                               mw1[:64], mw1[64:], b2(mb1), mw2v, b2(mb2v),
                               mw3v, b2(mb3v), b2(ms_), b2(mt_),
                               dw0, b2(db0), dw1, b2(db1), dw2p, db2p)

    out = outp[:, :OUT][None]
    nz = jnp.asarray(NZF, jnp.float32)
    return (loss[0, 0], out, nz)


# speedup vs baseline: 1.2262x; 1.2262x over previous
import functools

import jax
import jax.numpy as jnp
from jax import lax
from jax.experimental import pallas as pl
from jax.experimental.pallas import tpu as pltpu
from jax.experimental.pallas import tpu_sc as plsc

N = 10000
E = 160000
OUT = 3
NZF = float(N * OUT)

NC, NS = 2, 16
NW = NC * NS
CH = 128
EPAD = 163840
HALF = EPAD // 2
EPW2 = HALF // NW
NCH2 = EPW2 // CH
ACC = 10240
RPT = ACC // NS

NB = 1000
EB = 2048


def _ln(h, s, t):
    m = jnp.mean(h, axis=-1, keepdims=True)
    v = jnp.mean((h - m) ** 2, axis=-1, keepdims=True)
    return (h - m) / jnp.sqrt(v + 1e-5) * s + t


def _full(shape):
    return pl.BlockSpec(shape, lambda i: (0,) * len(shape))


def _encode_body(xin, w0, b0, w1, b1, w2, b2, s, t, out):
    x = xin[...]
    h = jnp.maximum(x @ w0[...] + b0[...], 0.0)
    h = jnp.maximum(h @ w1[...] + b1[...], 0.0)
    h = h @ w2[...] + b2[...]
    xln = _ln(h, s[...], t[...])
    pos = x[:, 124:127]
    out[...] = jnp.concatenate(
        [xln, pos, jnp.zeros((xln.shape[0], 61), jnp.float32)], axis=-1)


def _make_encode():
    return pl.pallas_call(
        _encode_body,
        grid=(N // NB,),
        in_specs=[pl.BlockSpec((NB, 128), lambda i: (i, 0))]
        + [_full(s) for s in [(128, 64), (1, 64), (64, 64), (1, 64),
                              (64, 64), (1, 64), (1, 64), (1, 64)]],
        out_specs=pl.BlockSpec((NB, 128), lambda i: (i, 0)),
        out_shape=jax.ShapeDtypeStruct((N, 128), jnp.float32),
    )


def _edge0_body(base, gs_, gd_,
                we0p, wdist, be0, we1, be1, we2, be2, es, et,
                mwe, mwxs, mwxd, mb1, mw2, mb2, mw3, mb3, ms, mt,
                out):
    gs = gs_[...]
    gd = gd_[...]
    xs, xd = gs[:, :64], gd[:, :64]
    d = gd[:, 64:80] - gs[:, 64:80]
    dist = jnp.sqrt(jnp.sum(d * d, axis=-1, keepdims=True) + 1e-12)
    h = d @ we0p[...] + dist * wdist[...] + be0[...]
    h = jnp.maximum(h, 0.0)
    h = jnp.maximum(h @ we1[...] + be1[...], 0.0)
    e0 = _ln(h @ we2[...] + be2[...], es[...], et[...])
    h = e0 @ mwe[...] + xs @ mwxs[...] + xd @ mwxd[...] + mb1[...]
    h = jnp.maximum(h, 0.0)
    h = jnp.maximum(h @ mw2[...] + mb2[...], 0.0)
    ef = _ln(h @ mw3[...] + mb3[...], ms[...], mt[...])
    e1 = e0 + ef
    row = base + pl.program_id(0) * EB + lax.broadcasted_iota(
        jnp.int32, (EB, 1), 0)
    e1 = jnp.concatenate([e1, jnp.zeros((EB, 64), jnp.float32)], axis=-1)
    out[...] = jnp.where(row < E, e1, 0.0)


@functools.cache
def _make_edge0(half):
    wshapes = [(16, 64), (1, 64), (1, 64), (64, 64), (1, 64), (64, 64),
               (1, 64), (1, 64), (1, 64),
               (64, 64), (64, 64), (64, 64), (1, 64), (64, 64), (1, 64),
               (64, 64), (1, 64), (1, 64), (1, 64)]
    return pl.pallas_call(
        functools.partial(_edge0_body, half * HALF),
        grid=(HALF // EB,),
        in_specs=[pl.BlockSpec((EB, 128), lambda i: (i, 0)),
                  pl.BlockSpec((EB, 128), lambda i: (i, 0))]
        + [_full(s) for s in wshapes],
        out_specs=pl.BlockSpec((EB, 128), lambda i: (i, 0)),
        out_shape=jax.ShapeDtypeStruct((HALF, 128), jnp.float32),
    )


def _edge1_body(base, e, gs_, gd_, mwe, mwxs, mwxd, mb1, mw2, mb2, mw3, mb3,
                ms, mt, out):
    ein = e[...][:, :64]
    xs, xd = gs_[...][:, :64], gd_[...][:, :64]
    h = ein @ mwe[...] + xs @ mwxs[...] + xd @ mwxd[...] + mb1[...]
    h = jnp.maximum(h, 0.0)
    h = jnp.maximum(h @ mw2[...] + mb2[...], 0.0)
    ef = _ln(h @ mw3[...] + mb3[...], ms[...], mt[...])
    e1 = ein + ef
    row = base + pl.program_id(0) * EB + lax.broadcasted_iota(
        jnp.int32, (EB, 1), 0)
    e1 = jnp.concatenate([e1, jnp.zeros((EB, 64), jnp.float32)], axis=-1)
    out[...] = jnp.where(row < E, e1, 0.0)


@functools.cache
def _make_edge1(half):
    wshapes = [(64, 64), (64, 64), (64, 64), (1, 64), (64, 64), (1, 64),
               (64, 64), (1, 64), (1, 64), (1, 64)]
    return pl.pallas_call(
        functools.partial(_edge1_body, half * HALF),
        grid=(HALF // EB,),
        in_specs=[pl.BlockSpec((EB, 128), lambda i: (i, 0)),
                  pl.BlockSpec((EB, 128), lambda i: (i, 0)),
                  pl.BlockSpec((EB, 128), lambda i: (i, 0))]
        + [_full(s) for s in wshapes],
        out_specs=pl.BlockSpec((EB, 128), lambda i: (i, 0)),
        out_shape=jax.ShapeDtypeStruct((HALF, 128), jnp.float32),
    )


def _agg_specs():
    return [pl.BlockSpec((1, NB, 128), lambda i: (0, i, 0)),
            pl.BlockSpec((1, NB, 128), lambda i: (1, i, 0)),
            pl.BlockSpec((1, NB, 128), lambda i: (0, i, 0)),
            pl.BlockSpec((1, NB, 128), lambda i: (1, i, 0))]


def _node0_body(x, a0, a1, a2, a3, wx, wa, b1, w2, b2, w3, b3, s, t, out):
    xin = x[...][:, :64]
    a = (a0[...][0] + a1[...][0] + a2[...][0] + a3[...][0])[:, :64]
    h = jnp.maximum(xin @ wx[...] + a @ wa[...] + b1[...], 0.0)
    h = jnp.maximum(h @ w2[...] + b2[...], 0.0)
    nf = _ln(h @ w3[...] + b3[...], s[...], t[...])
    x1 = xin + nf
    out[...] = jnp.concatenate(
        [x1, jnp.zeros((x1.shape[0], 64), jnp.float32)], axis=-1)


def _make_node0():
    wshapes = [(64, 64), (64, 64), (1, 64), (64, 64), (1, 64), (64, 64),
               (1, 64), (1, 64), (1, 64)]
    return pl.pallas_call(
        _node0_body,
        grid=(N // NB,),
        in_specs=[pl.BlockSpec((NB, 128), lambda i: (i, 0))]
        + _agg_specs()
        + [_full(s) for s in wshapes],
        out_specs=pl.BlockSpec((NB, 128), lambda i: (i, 0)),
        out_shape=jax.ShapeDtypeStruct((N, 128), jnp.float32),
    )


def _node1_body(x, a0, a1, a2, a3, tar,
                wx, wa, b1, w2, b2, w3, b3, s, t,
                dw0, db0, dw1, db1, dw2p, db2p,
                outp, loss):
    xin = x[...][:, :64]
    a = (a0[...][0] + a1[...][0] + a2[...][0] + a3[...][0])[:, :64]
    h = jnp.maximum(xin @ wx[...] + a @ wa[...] + b1[...], 0.0)
    h = jnp.maximum(h @ w2[...] + b2[...], 0.0)
    nf = _ln(h @ w3[...] + b3[...], s[...], t[...])
    x2 = xin + nf
    h = jnp.maximum(x2 @ dw0[...] + db0[...], 0.0)
    h = jnp.maximum(h @ dw1[...] + db1[...], 0.0)
    o = h @ dw2p[...] + db2p[...]
    outp[...] = o
    bs = jnp.sum((o - tar[...]) ** 2)
    i = pl.program_id(0)
    acc = jnp.where(i == 0, 0.0, loss[...]) + bs
    loss[...] = jnp.where(i == pl.num_programs(0) - 1, acc / NZF, acc)


def _make_node1():
    wshapes = [(64, 64), (64, 64), (1, 64), (64, 64), (1, 64), (64, 64),
               (1, 64), (1, 64), (1, 64),
               (64, 64), (1, 64), (64, 64), (1, 64), (64, 128), (1, 128)]
    return pl.pallas_call(
        _node1_body,
        grid=(N // NB,),
        in_specs=[pl.BlockSpec((NB, 128), lambda i: (i, 0))]
        + _agg_specs()
        + [pl.BlockSpec((NB, 128), lambda i: (i, 0))]
        + [_full(s) for s in wshapes],
        out_specs=[pl.BlockSpec((NB, 128), lambda i: (i, 0)),
                   pl.BlockSpec((1, 1), lambda i: (0, 0))],
        out_shape=[jax.ShapeDtypeStruct((N, 128), jnp.float32),
                   jax.ShapeDtypeStruct((1, 1), jnp.float32)],
    )


@functools.cache
def _sc_mesh():
    return plsc.VectorSubcoreMesh(core_axis_name="c", subcore_axis_name="s",
                                  num_cores=NC, num_subcores=NS)


def _gather_x_body(half, src2, dst2, xt, oxs, oxd, idx_s, idx_d,
                   bxs0, bxd0, bxs1, bxd1, sr0, sw0, sr1, sw1):
    c = lax.axis_index("c")
    s = lax.axis_index("s")
    wid = s * NC + c
    chunk0 = wid * (2 * NCH2)
    hoff = half * NCH2
    pltpu.sync_copy(src2.at[pl.ds(chunk0, 2 * NCH2)], idx_s)
    pltpu.sync_copy(dst2.at[pl.ds(chunk0, 2 * NCH2)], idx_d)
    base = wid * EPW2

    bufs = ((bxs0, bxd0, sr0, sw0), (bxs1, bxd1, sr1, sw1))

    for b in range(2):
        bx, bd, sr, _ = bufs[b]
        pltpu.async_copy(xt.at[idx_s.at[hoff + b]], bx, sr)
        pltpu.async_copy(xt.at[idx_d.at[hoff + b]], bd, sr)

    @pl.loop(0, NCH2, step=2)
    def _(j0):
        for b in range(2):
            j = j0 + b
            bx, bd, sr, sw = bufs[b]
            pltpu.make_async_copy(xt.at[idx_s.at[hoff + j]], bx, sr).wait()
            pltpu.make_async_copy(xt.at[idx_d.at[hoff + j]], bd, sr).wait()
            r0 = base + j * CH
            pltpu.async_copy(bx, oxs.at[pl.ds(r0, CH)], sw)
            pltpu.async_copy(bd, oxd.at[pl.ds(r0, CH)], sw)

            @pl.when(j + 2 < NCH2)
            def _():
                pltpu.make_async_copy(bx, oxs.at[pl.ds(r0, CH)], sw).wait()
                pltpu.make_async_copy(bd, oxd.at[pl.ds(r0, CH)], sw).wait()
                pltpu.async_copy(xt.at[idx_s.at[hoff + j + 2]], bx, sr)
                pltpu.async_copy(xt.at[idx_d.at[hoff + j + 2]], bd, sr)

    for b in range(2):
        bx, bd, _, sw = bufs[b]
        pltpu.make_async_copy(bx, oxs.at[pl.ds(0, CH)], sw).wait()
        pltpu.make_async_copy(bd, oxd.at[pl.ds(0, CH)], sw).wait()


@functools.cache
def _gather_x(half):
    return pl.kernel(
        functools.partial(_gather_x_body, half),
        out_type=[jax.ShapeDtypeStruct((HALF, 128), jnp.float32),
                  jax.ShapeDtypeStruct((HALF, 128), jnp.float32)],
        mesh=_sc_mesh(),
        scratch_types=[pltpu.VMEM((2 * NCH2, CH), jnp.int32),
                       pltpu.VMEM((2 * NCH2, CH), jnp.int32),
                       pltpu.VMEM((CH, 128), jnp.float32),
                       pltpu.VMEM((CH, 128), jnp.float32),
                       pltpu.VMEM((CH, 128), jnp.float32),
                       pltpu.VMEM((CH, 128), jnp.float32),
                       pltpu.SemaphoreType.DMA,
                       pltpu.SemaphoreType.DMA,
                       pltpu.SemaphoreType.DMA,
                       pltpu.SemaphoreType.DMA],
    )


def _scatter_body(half, e2, dst2, zeros_hbm, out, idxc, ebuf, acc):
    c = lax.axis_index("c")
    s = lax.axis_index("s")
    wid = s * NC + c
    rows0 = s * RPT
    pltpu.sync_copy(zeros_hbm.at[pl.ds(rows0, RPT)], acc.at[pl.ds(rows0, RPT)])
    plsc.subcore_barrier()
    chunk0 = half * (HALF // CH) + wid * NCH2
    base = wid * EPW2

    def body(j, carry):
        pltpu.sync_copy(dst2.at[chunk0 + j], idxc)
        pltpu.sync_copy(e2.at[pl.ds(base + j * CH, CH)], ebuf)
        pltpu.sync_copy(ebuf, acc.at[idxc], add=True)
        return carry

    lax.fori_loop(0, NCH2, body, 0)
    plsc.subcore_barrier()
    pltpu.sync_copy(acc.at[pl.ds(rows0, RPT)], out.at[c, pl.ds(rows0, RPT)])


@functools.cache
def _scatter(half):
    return pl.kernel(
        functools.partial(_scatter_body, half),
        out_type=jax.ShapeDtypeStruct((NC, ACC, 128), jnp.float32),
        mesh=_sc_mesh(),
        scratch_types=[pltpu.VMEM((CH,), jnp.int32),
                       pltpu.VMEM((CH, 128), jnp.float32),
                       pltpu.VMEM_SHARED((ACC, 128), jnp.float32)],
    )


def kernel(m_idx, m_gs, node_in, node_tar, params):
    x0in = node_in[0]
    tar = node_tar[0]
    src = m_gs[0, 0].astype(jnp.int32)
    dst = m_gs[0, 1].astype(jnp.int32)
    src2 = jnp.pad(src, (0, EPAD - E)).reshape(EPAD // CH, CH)
    dst2 = jnp.pad(dst, (0, EPAD - E)).reshape(EPAD // CH, CH)
    def _wmaj(a):
        return (a.reshape(2, NW, NCH2, CH).transpose(1, 0, 2, 3)
                .reshape(EPAD // CH, CH))
    src2g, dst2g = _wmaj(src2), _wmaj(dst2)

    tar_pad = jnp.pad(tar, ((0, 0), (0, 125)))
    zeros_acc = jnp.zeros((ACC, 128), jnp.float32)

    p = params

    def lyr(mp, i):
        return mp["layers"][i]

    def b2(x):
        return x.reshape(1, -1)

    enc = p["encode"]
    (ew0, eb0), (ew1, eb1), (ew2, eb2) = enc["layers"]
    es, et = enc["ln"]
    x0 = _make_encode()(x0in, ew0, b2(eb0), ew1, b2(eb1), ew2, b2(eb2),
                        b2(es), b2(et))

    ee = p["edge_enc"]
    (gw0, gb0), (gw1, gb1), (gw2, gb2) = ee["layers"]
    gs, gt = ee["ln"]
    we0p = jnp.pad(gw0[:3], ((0, 13), (0, 0)))
    wdist = gw0[3:4]
    m0 = lyr(p, 0)["edge"]
    (aw1, ab1), (aw2, ab2), (aw3, ab3) = m0["layers"]
    as_, at_ = m0["ln"]

    def edge0(h, gsh, gdh):
        return _make_edge0(h)(
            gsh, gdh,
            we0p, wdist, b2(gb0), gw1, b2(gb1), gw2, b2(gb2),
            b2(gs), b2(gt),
            aw1[:64], aw1[64:128], aw1[128:], b2(ab1),
            aw2, b2(ab2), aw3, b2(ab3), b2(as_), b2(at_))

    gsA, gdA = _gather_x(0)(src2g, dst2g, x0)
    gsB, gdB = _gather_x(1)(src2g, dst2g, x0)
    eA = edge0(0, gsA, gdA)
    eB = edge0(1, gsB, gdB)
    sA = _scatter(0)(eA, dst2, zeros_acc)
    sB = _scatter(1)(eB, dst2, zeros_acc)

    n0 = lyr(p, 0)["node"]
    (nw1, nb1), (nw2, nb2), (nw3, nb3) = n0["layers"]
    ns_, nt_ = n0["ln"]
    x1 = _make_node0()(x0, sA, sA, sB, sB,
                       nw1[:64], nw1[64:], b2(nb1), nw2, b2(nb2),
                       nw3, b2(nb3), b2(ns_), b2(nt_))

    m1 = lyr(p, 1)["edge"]
    (cw1, cb1), (cw2, cb2), (cw3, cb3) = m1["layers"]
    cs_, ct_ = m1["ln"]

    def edge1(h, eh, gsh, gdh):
        return _make_edge1(h)(
            eh, gsh, gdh,
            cw1[:64], cw1[64:128], cw1[128:], b2(cb1),
            cw2, b2(cb2), cw3, b2(cb3), b2(cs_), b2(ct_))

    gs1A, gd1A = _gather_x(0)(src2g, dst2g, x1)
    gs1B, gd1B = _gather_x(1)(src2g, dst2g, x1)
    e2A = edge1(0, eA, gs1A, gd1A)
    e2B = edge1(1, eB, gs1B, gd1B)
    s2A = _scatter(0)(e2A, dst2, zeros_acc)
    s2B = _scatter(1)(e2B, dst2, zeros_acc)

    n1 = lyr(p, 1)["node"]
    (mw1, mb1), (mw2v, mb2v), (mw3v, mb3v) = n1["layers"]
    ms_, mt_ = n1["ln"]
    dec = p["decode"]
    (dw0, db0), (dw1, db1), (dw2, db2v) = dec["layers"]
    dw2p = jnp.pad(dw2, ((0, 0), (0, 125)))
    db2p = jnp.pad(db2v.reshape(1, -1), ((0, 0), (0, 125)))
    outp, loss = _make_node1()(x1, s2A, s2A, s2B, s2B, tar_pad,
                               mw1[:64], mw1[64:], b2(mb1), mw2v, b2(mb2v),
                               mw3v, b2(mb3v), b2(ms_), b2(mt_),
                               dw0, b2(db0), dw1, b2(db1), dw2p, db2p)

    out = outp[:, :OUT][None]
    nz = jnp.asarray(NZF, jnp.float32)
    return (loss[0, 0], out, nz)


# speedup vs baseline: 1.2626x; 1.0297x over previous
import functools

import jax
import jax.numpy as jnp
from jax import lax
from jax.experimental import pallas as pl
from jax.experimental.pallas import tpu as pltpu
from jax.experimental.pallas import tpu_sc as plsc

N = 10000
E = 160000
OUT = 3
NZF = float(N * OUT)

NC, NS = 2, 16
NW = NC * NS
CH = 128
EPAD = 163840
HALF = EPAD // 2
EPW2 = HALF // NW
NCH2 = EPW2 // CH
ACC = 10240
RPT = ACC // NS

NB = 1000
EB = 2048


def _ln(h, s, t):
    m = jnp.mean(h, axis=-1, keepdims=True)
    v = jnp.mean((h - m) ** 2, axis=-1, keepdims=True)
    return (h - m) / jnp.sqrt(v + 1e-5) * s + t


def _full(shape):
    return pl.BlockSpec(shape, lambda i: (0,) * len(shape))


def _encode_body(xin, w0, b0, w1, b1, w2, b2, s, t, out):
    x = xin[...]
    h = jnp.maximum(x @ w0[...] + b0[...], 0.0)
    h = jnp.maximum(h @ w1[...] + b1[...], 0.0)
    h = h @ w2[...] + b2[...]
    xln = _ln(h, s[...], t[...])
    pos = x[:, 124:127]
    out[...] = jnp.concatenate(
        [xln, pos, jnp.zeros((xln.shape[0], 61), jnp.float32)], axis=-1)


def _make_encode():
    return pl.pallas_call(
        _encode_body,
        grid=(N // NB,),
        in_specs=[pl.BlockSpec((NB, 128), lambda i: (i, 0))]
        + [_full(s) for s in [(128, 64), (1, 64), (64, 64), (1, 64),
                              (64, 64), (1, 64), (1, 64), (1, 64)]],
        out_specs=pl.BlockSpec((NB, 128), lambda i: (i, 0)),
        out_shape=jax.ShapeDtypeStruct((N, 128), jnp.float32),
    )


def _edge0_body(base, gs_, gd_,
                we0p, wdist, be0, we1, be1, we2, be2, es, et,
                mwe, mwxs, mwxd, mb1, mw2, mb2, mw3, mb3, ms, mt,
                out):
    gs = gs_[...]
    gd = gd_[...]
    xs, xd = gs[:, :64], gd[:, :64]
    d = gd[:, 64:80] - gs[:, 64:80]
    dist = jnp.sqrt(jnp.sum(d * d, axis=-1, keepdims=True) + 1e-12)
    h = d @ we0p[...] + dist * wdist[...] + be0[...]
    h = jnp.maximum(h, 0.0)
    h = jnp.maximum(h @ we1[...] + be1[...], 0.0)
    e0 = _ln(h @ we2[...] + be2[...], es[...], et[...])
    h = e0 @ mwe[...] + xs @ mwxs[...] + xd @ mwxd[...] + mb1[...]
    h = jnp.maximum(h, 0.0)
    h = jnp.maximum(h @ mw2[...] + mb2[...], 0.0)
    ef = _ln(h @ mw3[...] + mb3[...], ms[...], mt[...])
    e1 = e0 + ef
    row = base + pl.program_id(0) * EB + lax.broadcasted_iota(
        jnp.int32, (EB, 1), 0)
    e1 = jnp.concatenate([e1, jnp.zeros((EB, 64), jnp.float32)], axis=-1)
    out[...] = jnp.where(row < E, e1, 0.0)


@functools.cache
def _make_edge0(half):
    wshapes = [(16, 64), (1, 64), (1, 64), (64, 64), (1, 64), (64, 64),
               (1, 64), (1, 64), (1, 64),
               (64, 64), (64, 64), (64, 64), (1, 64), (64, 64), (1, 64),
               (64, 64), (1, 64), (1, 64), (1, 64)]
    return pl.pallas_call(
        functools.partial(_edge0_body, half * HALF),
        grid=(HALF // EB,),
        in_specs=[pl.BlockSpec((EB, 128), lambda i: (i, 0)),
                  pl.BlockSpec((EB, 128), lambda i: (i, 0))]
        + [_full(s) for s in wshapes],
        out_specs=pl.BlockSpec((EB, 128), lambda i: (i, 0)),
        out_shape=jax.ShapeDtypeStruct((HALF, 128), jnp.float32),
    )


def _edge1_body(base, e, gs_, gd_, mwe, mwxs, mwxd, mb1, mw2, mb2, mw3, mb3,
                ms, mt, out):
    ein = e[...][:, :64]
    xs, xd = gs_[...][:, :64], gd_[...][:, :64]
    h = ein @ mwe[...] + xs @ mwxs[...] + xd @ mwxd[...] + mb1[...]
    h = jnp.maximum(h, 0.0)
    h = jnp.maximum(h @ mw2[...] + mb2[...], 0.0)
    ef = _ln(h @ mw3[...] + mb3[...], ms[...], mt[...])
    e1 = ein + ef
    row = base + pl.program_id(0) * EB + lax.broadcasted_iota(
        jnp.int32, (EB, 1), 0)
    e1 = jnp.concatenate([e1, jnp.zeros((EB, 64), jnp.float32)], axis=-1)
    out[...] = jnp.where(row < E, e1, 0.0)


@functools.cache
def _make_edge1(half):
    wshapes = [(64, 64), (64, 64), (64, 64), (1, 64), (64, 64), (1, 64),
               (64, 64), (1, 64), (1, 64), (1, 64)]
    return pl.pallas_call(
        functools.partial(_edge1_body, half * HALF),
        grid=(HALF // EB,),
        in_specs=[pl.BlockSpec((EB, 128), lambda i: (i, 0)),
                  pl.BlockSpec((EB, 128), lambda i: (i, 0)),
                  pl.BlockSpec((EB, 128), lambda i: (i, 0))]
        + [_full(s) for s in wshapes],
        out_specs=pl.BlockSpec((EB, 128), lambda i: (i, 0)),
        out_shape=jax.ShapeDtypeStruct((HALF, 128), jnp.float32),
    )


def _agg_specs():
    return [pl.BlockSpec((1, NB, 128), lambda i: (0, i, 0)),
            pl.BlockSpec((1, NB, 128), lambda i: (1, i, 0)),
            pl.BlockSpec((1, NB, 128), lambda i: (0, i, 0)),
            pl.BlockSpec((1, NB, 128), lambda i: (1, i, 0))]


def _node0_body(x, a0, a1, a2, a3, wx, wa, b1, w2, b2, w3, b3, s, t, out):
    xin = x[...][:, :64]
    a = (a0[...][0] + a1[...][0] + a2[...][0] + a3[...][0])[:, :64]
    h = jnp.maximum(xin @ wx[...] + a @ wa[...] + b1[...], 0.0)
    h = jnp.maximum(h @ w2[...] + b2[...], 0.0)
    nf = _ln(h @ w3[...] + b3[...], s[...], t[...])
    x1 = xin + nf
    out[...] = jnp.concatenate(
        [x1, jnp.zeros((x1.shape[0], 64), jnp.float32)], axis=-1)


def _make_node0():
    wshapes = [(64, 64), (64, 64), (1, 64), (64, 64), (1, 64), (64, 64),
               (1, 64), (1, 64), (1, 64)]
    return pl.pallas_call(
        _node0_body,
        grid=(N // NB,),
        in_specs=[pl.BlockSpec((NB, 128), lambda i: (i, 0))]
        + _agg_specs()
        + [_full(s) for s in wshapes],
        out_specs=pl.BlockSpec((NB, 128), lambda i: (i, 0)),
        out_shape=jax.ShapeDtypeStruct((N, 128), jnp.float32),
    )


def _node1_body(x, a0, a1, a2, a3, tar,
                wx, wa, b1, w2, b2, w3, b3, s, t,
                dw0, db0, dw1, db1, dw2p, db2p,
                outp, loss):
    xin = x[...][:, :64]
    a = (a0[...][0] + a1[...][0] + a2[...][0] + a3[...][0])[:, :64]
    h = jnp.maximum(xin @ wx[...] + a @ wa[...] + b1[...], 0.0)
    h = jnp.maximum(h @ w2[...] + b2[...], 0.0)
    nf = _ln(h @ w3[...] + b3[...], s[...], t[...])
    x2 = xin + nf
    h = jnp.maximum(x2 @ dw0[...] + db0[...], 0.0)
    h = jnp.maximum(h @ dw1[...] + db1[...], 0.0)
    o = h @ dw2p[...] + db2p[...]
    outp[...] = o
    bs = jnp.sum((o - tar[...]) ** 2)
    i = pl.program_id(0)
    acc = jnp.where(i == 0, 0.0, loss[...]) + bs
    loss[...] = jnp.where(i == pl.num_programs(0) - 1, acc / NZF, acc)


def _make_node1():
    wshapes = [(64, 64), (64, 64), (1, 64), (64, 64), (1, 64), (64, 64),
               (1, 64), (1, 64), (1, 64),
               (64, 64), (1, 64), (64, 64), (1, 64), (64, 128), (1, 128)]
    return pl.pallas_call(
        _node1_body,
        grid=(N // NB,),
        in_specs=[pl.BlockSpec((NB, 128), lambda i: (i, 0))]
        + _agg_specs()
        + [pl.BlockSpec((NB, 128), lambda i: (i, 0))]
        + [_full(s) for s in wshapes],
        out_specs=[pl.BlockSpec((NB, 128), lambda i: (i, 0)),
                   pl.BlockSpec((1, 1), lambda i: (0, 0))],
        out_shape=[jax.ShapeDtypeStruct((N, 128), jnp.float32),
                   jax.ShapeDtypeStruct((1, 1), jnp.float32)],
    )


@functools.cache
def _sc_mesh():
    return plsc.VectorSubcoreMesh(core_axis_name="c", subcore_axis_name="s",
                                  num_cores=NC, num_subcores=NS)


def _gather_x_body(half, src2, dst2, xt, oxs, oxd, idx_s, idx_d,
                   bxs0, bxd0, bxs1, bxd1, sr0, sw0, sr1, sw1):
    c = lax.axis_index("c")
    s = lax.axis_index("s")
    wid = s * NC + c
    chunk0 = wid * (2 * NCH2)
    hoff = half * NCH2
    pltpu.sync_copy(src2.at[pl.ds(chunk0, 2 * NCH2)], idx_s)
    pltpu.sync_copy(dst2.at[pl.ds(chunk0, 2 * NCH2)], idx_d)
    base = wid * EPW2

    bufs = ((bxs0, bxd0, sr0, sw0), (bxs1, bxd1, sr1, sw1))

    for b in range(2):
        bx, bd, sr, _ = bufs[b]
        pltpu.async_copy(xt.at[idx_s.at[hoff + b]], bx, sr)
        pltpu.async_copy(xt.at[idx_d.at[hoff + b]], bd, sr)

    @pl.loop(0, NCH2, step=2)
    def _(j0):
        for b in range(2):
            j = j0 + b
            bx, bd, sr, sw = bufs[b]
            pltpu.make_async_copy(xt.at[idx_s.at[hoff + j]], bx, sr).wait()
            pltpu.make_async_copy(xt.at[idx_d.at[hoff + j]], bd, sr).wait()
            r0 = base + j * CH
            pltpu.async_copy(bx, oxs.at[pl.ds(r0, CH)], sw)
            pltpu.async_copy(bd, oxd.at[pl.ds(r0, CH)], sw)

            @pl.when(j + 2 < NCH2)
            def _():
                pltpu.make_async_copy(bx, oxs.at[pl.ds(r0, CH)], sw).wait()
                pltpu.make_async_copy(bd, oxd.at[pl.ds(r0, CH)], sw).wait()
                pltpu.async_copy(xt.at[idx_s.at[hoff + j + 2]], bx, sr)
                pltpu.async_copy(xt.at[idx_d.at[hoff + j + 2]], bd, sr)

    for b in range(2):
        bx, bd, _, sw = bufs[b]
        pltpu.make_async_copy(bx, oxs.at[pl.ds(0, CH)], sw).wait()
        pltpu.make_async_copy(bd, oxd.at[pl.ds(0, CH)], sw).wait()


@functools.cache
def _gather_x(half):
    return pl.kernel(
        functools.partial(_gather_x_body, half),
        out_type=[jax.ShapeDtypeStruct((HALF, 128), jnp.float32),
                  jax.ShapeDtypeStruct((HALF, 128), jnp.float32)],
        mesh=_sc_mesh(),
        scratch_types=[pltpu.VMEM((2 * NCH2, CH), jnp.int32),
                       pltpu.VMEM((2 * NCH2, CH), jnp.int32),
                       pltpu.VMEM((CH, 128), jnp.float32),
                       pltpu.VMEM((CH, 128), jnp.float32),
                       pltpu.VMEM((CH, 128), jnp.float32),
                       pltpu.VMEM((CH, 128), jnp.float32),
                       pltpu.SemaphoreType.DMA,
                       pltpu.SemaphoreType.DMA,
                       pltpu.SemaphoreType.DMA,
                       pltpu.SemaphoreType.DMA],
    )


def _scatter_body(half, e2, dst2, zeros_hbm, out,
                  idx0, idx1, eb0, eb1, acc, sm0, sm1):
    c = lax.axis_index("c")
    s = lax.axis_index("s")
    wid = s * NC + c
    rows0 = s * RPT
    pltpu.sync_copy(zeros_hbm.at[pl.ds(rows0, RPT)], acc.at[pl.ds(rows0, RPT)])
    plsc.subcore_barrier()
    chunk0 = half * (HALF // CH) + wid * NCH2
    base = wid * EPW2

    bufs = ((idx0, eb0, sm0), (idx1, eb1, sm1))

    for b in range(2):
        idx, eb, sm = bufs[b]
        pltpu.async_copy(dst2.at[chunk0 + b], idx, sm)
        pltpu.async_copy(e2.at[pl.ds(base + b * CH, CH)], eb, sm)

    @pl.loop(0, NCH2, step=2)
    def _(j0):
        for b in range(2):
            j = j0 + b
            idx, eb, sm = bufs[b]
            pltpu.make_async_copy(dst2.at[chunk0 + j], idx, sm).wait()
            pltpu.make_async_copy(e2.at[pl.ds(base + j * CH, CH)], eb,
                                  sm).wait()
            pltpu.sync_copy(eb, acc.at[idx], add=True)

            @pl.when(j + 2 < NCH2)
            def _():
                pltpu.async_copy(dst2.at[chunk0 + j + 2], idx, sm)
                pltpu.async_copy(e2.at[pl.ds(base + (j + 2) * CH, CH)], eb, sm)

    plsc.subcore_barrier()
    pltpu.sync_copy(acc.at[pl.ds(rows0, RPT)], out.at[c, pl.ds(rows0, RPT)])


@functools.cache
def _scatter(half):
    return pl.kernel(
        functools.partial(_scatter_body, half),
        out_type=jax.ShapeDtypeStruct((NC, ACC, 128), jnp.float32),
        mesh=_sc_mesh(),
        scratch_types=[pltpu.VMEM((CH,), jnp.int32),
                       pltpu.VMEM((CH,), jnp.int32),
                       pltpu.VMEM((CH, 128), jnp.float32),
                       pltpu.VMEM((CH, 128), jnp.float32),
                       pltpu.VMEM_SHARED((ACC, 128), jnp.float32),
                       pltpu.SemaphoreType.DMA,
                       pltpu.SemaphoreType.DMA],
    )


def kernel(m_idx, m_gs, node_in, node_tar, params):
    x0in = node_in[0]
    tar = node_tar[0]
    src = m_gs[0, 0].astype(jnp.int32)
    dst = m_gs[0, 1].astype(jnp.int32)
    src2 = jnp.pad(src, (0, EPAD - E)).reshape(EPAD // CH, CH)
    dst2 = jnp.pad(dst, (0, EPAD - E)).reshape(EPAD // CH, CH)
    def _wmaj(a):
        return (a.reshape(2, NW, NCH2, CH).transpose(1, 0, 2, 3)
                .reshape(EPAD // CH, CH))
    src2g, dst2g = _wmaj(src2), _wmaj(dst2)

    tar_pad = jnp.pad(tar, ((0, 0), (0, 125)))
    zeros_acc = jnp.zeros((ACC, 128), jnp.float32)

    p = params

    def lyr(mp, i):
        return mp["layers"][i]

    def b2(x):
        return x.reshape(1, -1)

    enc = p["encode"]
    (ew0, eb0), (ew1, eb1), (ew2, eb2) = enc["layers"]
    es, et = enc["ln"]
    x0 = _make_encode()(x0in, ew0, b2(eb0), ew1, b2(eb1), ew2, b2(eb2),
                        b2(es), b2(et))

    ee = p["edge_enc"]
    (gw0, gb0), (gw1, gb1), (gw2, gb2) = ee["layers"]
    gs, gt = ee["ln"]
    we0p = jnp.pad(gw0[:3], ((0, 13), (0, 0)))
    wdist = gw0[3:4]
    m0 = lyr(p, 0)["edge"]
    (aw1, ab1), (aw2, ab2), (aw3, ab3) = m0["layers"]
    as_, at_ = m0["ln"]

    def edge0(h, gsh, gdh):
        return _make_edge0(h)(
            gsh, gdh,
            we0p, wdist, b2(gb0), gw1, b2(gb1), gw2, b2(gb2),
            b2(gs), b2(gt),
            aw1[:64], aw1[64:128], aw1[128:], b2(ab1),
            aw2, b2(ab2), aw3, b2(ab3), b2(as_), b2(at_))

    gsA, gdA = _gather_x(0)(src2g, dst2g, x0)
    gsB, gdB = _gather_x(1)(src2g, dst2g, x0)
    eA = edge0(0, gsA, gdA)
    eB = edge0(1, gsB, gdB)
    sA = _scatter(0)(eA, dst2, zeros_acc)
    sB = _scatter(1)(eB, dst2, zeros_acc)

    n0 = lyr(p, 0)["node"]
    (nw1, nb1), (nw2, nb2), (nw3, nb3) = n0["layers"]
    ns_, nt_ = n0["ln"]
    x1 = _make_node0()(x0, sA, sA, sB, sB,
                       nw1[:64], nw1[64:], b2(nb1), nw2, b2(nb2),
                       nw3, b2(nb3), b2(ns_), b2(nt_))

    m1 = lyr(p, 1)["edge"]
    (cw1, cb1), (cw2, cb2), (cw3, cb3) = m1["layers"]
    cs_, ct_ = m1["ln"]

    def edge1(h, eh, gsh, gdh):
        return _make_edge1(h)(
            eh, gsh, gdh,
            cw1[:64], cw1[64:128], cw1[128:], b2(cb1),
            cw2, b2(cb2), cw3, b2(cb3), b2(cs_), b2(ct_))

    gs1A, gd1A = _gather_x(0)(src2g, dst2g, x1)
    gs1B, gd1B = _gather_x(1)(src2g, dst2g, x1)
    e2A = edge1(0, eA, gs1A, gd1A)
    e2B = edge1(1, eB, gs1B, gd1B)
    s2A = _scatter(0)(e2A, dst2, zeros_acc)
    s2B = _scatter(1)(e2B, dst2, zeros_acc)

    n1 = lyr(p, 1)["node"]
    (mw1, mb1), (mw2v, mb2v), (mw3v, mb3v) = n1["layers"]
    ms_, mt_ = n1["ln"]
    dec = p["decode"]
    (dw0, db0), (dw1, db1), (dw2, db2v) = dec["layers"]
    dw2p = jnp.pad(dw2, ((0, 0), (0, 125)))
    db2p = jnp.pad(db2v.reshape(1, -1), ((0, 0), (0, 125)))
    outp, loss = _make_node1()(x1, s2A, s2A, s2B, s2B, tar_pad,
                               mw1[:64], mw1[64:], b2(mb1), mw2v, b2(mb2v),
                               mw3v, b2(mb3v), b2(ms_), b2(mt_),
                               dw0, b2(db0), dw1, b2(db1), dw2p, db2p)

    out = outp[:, :OUT][None]
    nz = jnp.asarray(NZF, jnp.float32)
    return (loss[0, 0], out, nz)
